# baseline (speedup 1.0000x reference)
import functools

import jax
import jax.numpy as jnp
from jax import lax
from jax.experimental import pallas as pl
from jax.experimental.pallas import tpu as pltpu
from jax.experimental.pallas import tpu_sc as plsc

_H, _B, _N, _D, _C = 1, 64, 576, 256, 1024
_ROWS = _B * _N
_TILE = 256
_NSPLIT = 3
_RPC = _ROWS // _NSPLIT
_TPC = _RPC // _TILE

_NC, _NS = 2, 16
_NW = _NC * _NS
_BPW = _RPC // _NW
_CHUNK = 192
_NCHUNK = _BPW // _CHUNK

_SC_MESH = plsc.VectorSubcoreMesh(core_axis_name="c", subcore_axis_name="s")


def _dist_argmax_body(x_ref, e_ref, *rest):
    dist_ref, ind_ref = rest[-2], rest[-1]
    x = x_ref[...]
    e = e_ref[...]
    dist = jax.lax.dot_general(
        x, e, (((1,), (1,)), ((), ())), preferred_element_type=jnp.float32)
    dist_ref[...] = dist
    ind_ref[0, 0, :] = jnp.argmax(dist, axis=1).astype(jnp.int32)


def _make_tc_call(k):
    first = k == 0
    in_specs = [
        pl.BlockSpec((_TILE, _D), lambda i: (i, 0)),
        pl.BlockSpec((_C, _D), lambda i: (0, 0)),
    ]
    if not first:
        in_specs.append(pl.BlockSpec(memory_space=pl.ANY))
    return pl.pallas_call(
        _dist_argmax_body,
        grid=(_TPC,),
        in_specs=in_specs,
        out_specs=[
            pl.BlockSpec((_TILE, _C), lambda i, k=k: (i + k * _TPC, 0)),
            pl.BlockSpec((1, 1, _TILE), lambda i: (i, 0, 0)),
        ],
        out_shape=[
            jax.ShapeDtypeStruct((_ROWS, _C), jnp.float32),
            jax.ShapeDtypeStruct((_TPC, 1, _TILE), jnp.int32),
        ],
        input_output_aliases={} if first else {2: 0},
    )


_TC_CALLS = [_make_tc_call(k) for k in range(_NSPLIT)]


def _sc_gather_body(row_lo, ind_hbm, table_hbm, out_hbm,
                    idx_v, rows0, rows1, sem0, sem1):
    wid = lax.axis_index("s") * _NC + lax.axis_index("c")
    lbase = wid * _BPW
    gbase = row_lo + lbase
    pltpu.sync_copy(ind_hbm.at[pl.ds(lbase, _BPW)], idx_v)
    bufs = (rows0, rows1)
    sems = (sem0, sem1)
    copies = []
    for i in range(_NCHUNK):
        cp = pltpu.async_copy(
            table_hbm.at[idx_v.at[pl.ds(i * _CHUNK, _CHUNK)]],
            bufs[i % 2], sems[i % 2])
        copies.append(cp)
        if i > 0:
            copies[i - 1].wait()
            pltpu.sync_copy(bufs[(i - 1) % 2],
                            out_hbm.at[pl.ds(gbase + (i - 1) * _CHUNK, _CHUNK)])
    copies[_NCHUNK - 1].wait()
    pltpu.sync_copy(bufs[(_NCHUNK - 1) % 2],
                    out_hbm.at[pl.ds(gbase + (_NCHUNK - 1) * _CHUNK, _CHUNK)])


_SC_SCRATCH = [
    pltpu.VMEM((_BPW,), jnp.int32),
    pltpu.VMEM((_CHUNK, _D), jnp.float32),
    pltpu.VMEM((_CHUNK, _D), jnp.float32),
    pltpu.SemaphoreType.DMA,
    pltpu.SemaphoreType.DMA,
]

_sc_gather_first = functools.partial(
    pl.kernel,
    mesh=_SC_MESH,
    out_type=jax.ShapeDtypeStruct((_ROWS, _D), jnp.float32),
    scratch_types=_SC_SCRATCH,
)(functools.partial(_sc_gather_body, 0))

_sc_gather_at = [
    functools.partial(
        pl.kernel,
        mesh=_SC_MESH,
        out_type=(),
        scratch_types=_SC_SCRATCH,
    )(functools.partial(_sc_gather_body, k * _RPC))
    for k in range(1, _NSPLIT)
]


def kernel(x, embed):
    x = x.astype(jnp.float32)
    xf = x.reshape(_ROWS, _D)
    e = embed.reshape(_C, _D)

    dist = None
    inds = []
    quant_ref = None
    for k in range(_NSPLIT):
        xk = lax.slice_in_dim(xf, k * _RPC, (k + 1) * _RPC, axis=0)
        if k == 0:
            dist, ind3 = _TC_CALLS[0](xk, e)
        else:
            dist, ind3 = _TC_CALLS[k](xk, e, dist)
        inds.append(ind3)
        ind_flat = ind3.reshape(_RPC)
        if k == 0:
            quant0 = _sc_gather_first(ind_flat, e)
            quant_ref = jax.new_ref(quant0)
        else:
            _sc_gather_at[k - 1](ind_flat, e, quant_ref)

    quant = quant_ref[...]
    quantize = quant.reshape(_B, _N, _D)
    embed_ind = jnp.concatenate(
        [i3.reshape(_RPC) for i3 in inds]).reshape(_B, _N)
    dist_out = dist.reshape(_H, _B, _N, _C)
    return (quantize, embed_ind, dist_out)


# speedup vs baseline: 1.4801x; 1.4801x over previous
import jax
import jax.numpy as jnp
from jax.experimental import pallas as pl

_H, _B, _N, _D, _C = 1, 64, 576, 256, 1024
_ROWS = _B * _N
_TILE = 512
_GRID = _ROWS // _TILE


def _vq_kernel(x_ref, e_ref, dist_ref, ind_ref, quant_ref):
    x = x_ref[...]
    e = e_ref[...]
    dist = jax.lax.dot_general(
        x, e, (((1,), (1,)), ((), ())), preferred_element_type=jnp.float32)
    dist_ref[...] = dist
    ind = jnp.argmax(dist, axis=1).astype(jnp.int32)
    ind_ref[0, 0, :] = ind
    col = jax.lax.broadcasted_iota(jnp.int32, (_TILE, _C), 1)
    onehot = (col == ind[:, None]).astype(jnp.float32)
    quant_ref[...] = jax.lax.dot_general(
        onehot, e, (((1,), (0,)), ((), ())), preferred_element_type=jnp.float32)


def kernel(x, embed):
    x = x.astype(jnp.float32)
    xf = x.reshape(_ROWS, _D)
    e = embed.reshape(_C, _D)
    dist, ind3, quant = pl.pallas_call(
        _vq_kernel,
        grid=(_GRID,),
        in_specs=[
            pl.BlockSpec((_TILE, _D), lambda i: (i, 0)),
            pl.BlockSpec((_C, _D), lambda i: (0, 0)),
        ],
        out_specs=[
            pl.BlockSpec((_TILE, _C), lambda i: (i, 0)),
            pl.BlockSpec((1, 1, _TILE), lambda i: (i, 0, 0)),
            pl.BlockSpec((_TILE, _D), lambda i: (i, 0)),
        ],
        out_shape=[
            jax.ShapeDtypeStruct((_ROWS, _C), jnp.float32),
            jax.ShapeDtypeStruct((_GRID, 1, _TILE), jnp.int32),
            jax.ShapeDtypeStruct((_ROWS, _D), jnp.float32),
        ],
    )(xf, e)
    quantize = quant.reshape(_B, _N, _D)
    embed_ind = ind3.reshape(_B, _N)
    dist_out = dist.reshape(_H, _B, _N, _C)
    return (quantize, embed_ind, dist_out)


# speedup vs baseline: 1.8052x; 1.2197x over previous
import jax
import jax.numpy as jnp
from jax.experimental import pallas as pl

_H, _B, _N, _D, _C = 1, 64, 576, 256, 1024
_ROWS = _B * _N
_TILE = 1024
_GRID = _ROWS // _TILE


def _vq_kernel(x_ref, e_ref, dist_ref, ind_ref, quant_ref):
    x = x_ref[...]
    e = e_ref[...]
    dist = jax.lax.dot_general(
        x, e, (((1,), (1,)), ((), ())), preferred_element_type=jnp.float32)
    dist_ref[...] = dist
    ind = jnp.argmax(dist, axis=1).astype(jnp.int32)
    ind_ref[0, 0, :] = ind
    col = jax.lax.broadcasted_iota(jnp.int32, (_TILE, _C), 1)
    onehot = (col == ind[:, None]).astype(jnp.float32)
    quant_ref[...] = jax.lax.dot_general(
        onehot, e, (((1,), (0,)), ((), ())), preferred_element_type=jnp.float32)


def kernel(x, embed):
    x = x.astype(jnp.float32)
    xf = x.reshape(_ROWS, _D)
    e = embed.reshape(_C, _D)
    dist, ind3, quant = pl.pallas_call(
        _vq_kernel,
        grid=(_GRID,),
        in_specs=[
            pl.BlockSpec((_TILE, _D), lambda i: (i, 0)),
            pl.BlockSpec((_C, _D), lambda i: (0, 0)),
        ],
        out_specs=[
            pl.BlockSpec((_TILE, _C), lambda i: (i, 0)),
            pl.BlockSpec((1, 1, _TILE), lambda i: (i, 0, 0)),
            pl.BlockSpec((_TILE, _D), lambda i: (i, 0)),
        ],
        out_shape=[
            jax.ShapeDtypeStruct((_ROWS, _C), jnp.float32),
            jax.ShapeDtypeStruct((_GRID, 1, _TILE), jnp.int32),
            jax.ShapeDtypeStruct((_ROWS, _D), jnp.float32),
        ],
    )(xf, e)
    quantize = quant.reshape(_B, _N, _D)
    embed_ind = ind3.reshape(_B, _N)
    dist_out = dist.reshape(_H, _B, _N, _C)
    return (quantize, embed_ind, dist_out)


# speedup vs baseline: 1.9902x; 1.1025x over previous
import jax
import jax.numpy as jnp
from jax.experimental import pallas as pl

_H, _B, _N, _D, _C = 1, 64, 576, 256, 1024
_ROWS = _B * _N
_TILE = 2048
_GRID = _ROWS // _TILE


def _vq_kernel(x_ref, e_ref, dist_ref, ind_ref, quant_ref):
    x = x_ref[...]
    e = e_ref[...]
    dist = jax.lax.dot_general(
        x, e, (((1,), (1,)), ((), ())), preferred_element_type=jnp.float32)
    dist_ref[...] = dist
    ind = jnp.argmax(dist, axis=1).astype(jnp.int32)
    ind_ref[0, 0, :] = ind
    col = jax.lax.broadcasted_iota(jnp.int32, (_TILE, _C), 1)
    onehot = (col == ind[:, None]).astype(jnp.float32)
    quant_ref[...] = jax.lax.dot_general(
        onehot, e, (((1,), (0,)), ((), ())), preferred_element_type=jnp.float32)


def kernel(x, embed):
    x = x.astype(jnp.float32)
    xf = x.reshape(_ROWS, _D)
    e = embed.reshape(_C, _D)
    dist, ind3, quant = pl.pallas_call(
        _vq_kernel,
        grid=(_GRID,),
        in_specs=[
            pl.BlockSpec((_TILE, _D), lambda i: (i, 0)),
            pl.BlockSpec((_C, _D), lambda i: (0, 0)),
        ],
        out_specs=[
            pl.BlockSpec((_TILE, _C), lambda i: (i, 0)),
            pl.BlockSpec((1, 1, _TILE), lambda i: (i, 0, 0)),
            pl.BlockSpec((_TILE, _D), lambda i: (i, 0)),
        ],
        out_shape=[
            jax.ShapeDtypeStruct((_ROWS, _C), jnp.float32),
            jax.ShapeDtypeStruct((_GRID, 1, _TILE), jnp.int32),
            jax.ShapeDtypeStruct((_ROWS, _D), jnp.float32),
        ],
    )(xf, e)
    quantize = quant.reshape(_B, _N, _D)
    embed_ind = ind3.reshape(_B, _N)
    dist_out = dist.reshape(_H, _B, _N, _C)
    return (quantize, embed_ind, dist_out)


# speedup vs baseline: 2.0661x; 1.0381x over previous
import jax
import jax.numpy as jnp
from jax.experimental import pallas as pl

_H, _B, _N, _D, _C = 1, 64, 576, 256, 1024
_ROWS = _B * _N
_TILE = 4096
_GRID = _ROWS // _TILE


def _vq_kernel(x_ref, e_ref, dist_ref, ind_ref, quant_ref):
    x = x_ref[...]
    e = e_ref[...]
    dist = jax.lax.dot_general(
        x, e, (((1,), (1,)), ((), ())), preferred_element_type=jnp.float32)
    dist_ref[...] = dist
    ind = jnp.argmax(dist, axis=1).astype(jnp.int32)
    ind_ref[0, 0, :] = ind
    col = jax.lax.broadcasted_iota(jnp.int32, (_TILE, _C), 1)
    onehot = (col == ind[:, None]).astype(jnp.float32)
    quant_ref[...] = jax.lax.dot_general(
        onehot, e, (((1,), (0,)), ((), ())), preferred_element_type=jnp.float32)


def kernel(x, embed):
    x = x.astype(jnp.float32)
    xf = x.reshape(_ROWS, _D)
    e = embed.reshape(_C, _D)
    dist, ind3, quant = pl.pallas_call(
        _vq_kernel,
        grid=(_GRID,),
        in_specs=[
            pl.BlockSpec((_TILE, _D), lambda i: (i, 0)),
            pl.BlockSpec((_C, _D), lambda i: (0, 0)),
        ],
        out_specs=[
            pl.BlockSpec((_TILE, _C), lambda i: (i, 0)),
            pl.BlockSpec((1, 1, _TILE), lambda i: (i, 0, 0)),
            pl.BlockSpec((_TILE, _D), lambda i: (i, 0)),
        ],
        out_shape=[
            jax.ShapeDtypeStruct((_ROWS, _C), jnp.float32),
            jax.ShapeDtypeStruct((_GRID, 1, _TILE), jnp.int32),
            jax.ShapeDtypeStruct((_ROWS, _D), jnp.float32),
        ],
    )(xf, e)
    quantize = quant.reshape(_B, _N, _D)
    embed_ind = ind3.reshape(_B, _N)
    dist_out = dist.reshape(_H, _B, _N, _C)
    return (quantize, embed_ind, dist_out)


# speedup vs baseline: 2.0687x; 1.0013x over previous
import jax
import jax.numpy as jnp
from jax.experimental import pallas as pl
from jax.experimental.pallas import tpu as pltpu

_H, _B, _N, _D, _C = 1, 64, 576, 256, 1024
_ROWS = _B * _N
_TILE = 4096
_GRID = _ROWS // _TILE


def _vq_kernel(x_ref, e_ref, dist_ref, ind_ref, quant_ref):
    x = x_ref[...]
    e = e_ref[...]
    dist = jax.lax.dot_general(
        x, e, (((1,), (1,)), ((), ())), preferred_element_type=jnp.float32)
    dist_ref[...] = dist
    ind = jnp.argmax(dist, axis=1).astype(jnp.int32)
    ind_ref[0, 0, :] = ind
    col = jax.lax.broadcasted_iota(jnp.int32, (_TILE, _C), 1)
    onehot = (col == ind[:, None]).astype(jnp.float32)
    quant_ref[...] = jax.lax.dot_general(
        onehot, e, (((1,), (0,)), ((), ())), preferred_element_type=jnp.float32)


def kernel(x, embed):
    x = x.astype(jnp.float32)
    xf = x.reshape(_ROWS, _D)
    e = embed.reshape(_C, _D)
    dist, ind3, quant = pl.pallas_call(
        _vq_kernel,
        grid=(_GRID,),
        in_specs=[
            pl.BlockSpec((_TILE, _D), lambda i: (i, 0)),
            pl.BlockSpec((_C, _D), lambda i: (0, 0)),
        ],
        out_specs=[
            pl.BlockSpec((_TILE, _C), lambda i: (i, 0)),
            pl.BlockSpec((1, 1, _TILE), lambda i: (i, 0, 0)),
            pl.BlockSpec((_TILE, _D), lambda i: (i, 0)),
        ],
        out_shape=[
            jax.ShapeDtypeStruct((_ROWS, _C), jnp.float32),
            jax.ShapeDtypeStruct((_GRID, 1, _TILE), jnp.int32),
            jax.ShapeDtypeStruct((_ROWS, _D), jnp.float32),
        ],
        compiler_params=pltpu.CompilerParams(
            vmem_limit_bytes=100 * 1024 * 1024),
    )(xf, e)
    quantize = quant.reshape(_B, _N, _D)
    embed_ind = ind3.reshape(_B, _N)
    dist_out = dist.reshape(_H, _B, _N, _C)
    return (quantize, embed_ind, dist_out)
